# baseline (speedup 1.0000x reference)
import jax
import jax.numpy as jnp
from jax.experimental import pallas as pl
from jax.experimental.pallas import tpu as pltpu

BATCH = 256
DIM_CODES = 8
DICT_SIZE = 1024
DIM_EMBED = 64


NBUF = 4


def _vq_kernel(mu_ref, dict_hbm_ref, cw_ref, oh_hbm_ref,
               d_buf, oh_buf, d_sems, oh_sems):
    half = DIM_CODES // 2
    d_pending = []
    for h in range(2):
        cp = pltpu.make_async_copy(
            dict_hbm_ref.at[pl.ds(h * half, half)], d_buf.at[h],
            d_sems.at[h])
        cp.start()
        d_pending.append(cp)

    oh_pending = [None] * NBUF
    for c in range(DIM_CODES):
        if c % half == 0:
            d_pending[c // half].wait()
        x = mu_ref[:, c * DIM_EMBED:(c + 1) * DIM_EMBED]
        d = d_buf[c // half, c % half]
        a2 = jnp.sum(x * x, axis=1, keepdims=True)
        b2 = jnp.sum(d * d, axis=1)[None, :]
        ab = jax.lax.dot_general(
            x, d, (((1,), (1,)), ((), ())),
            preferred_element_type=jnp.float32)
        dist = a2 - 2.0 * ab + b2
        idx = jnp.argmin(dist, axis=1)
        iota = jax.lax.broadcasted_iota(jnp.int32, (BATCH, DICT_SIZE), 1)
        hit = iota == idx[:, None].astype(jnp.int32)
        one_hot = hit.astype(jnp.float32)
        buf = c % NBUF
        if oh_pending[buf] is not None:
            oh_pending[buf].wait()
        oh_buf[buf] = one_hot
        cp = pltpu.make_async_copy(
            oh_buf.at[buf], oh_hbm_ref.at[:, c, :], oh_sems.at[buf])
        cp.start()
        oh_pending[buf] = cp
        oh_bf = hit.astype(jnp.bfloat16)
        d1 = d.astype(jnp.bfloat16)
        d2 = (d - d1.astype(jnp.float32)).astype(jnp.bfloat16)
        dims = (((1,), (0,)), ((), ()))
        cw = jax.lax.dot_general(oh_bf, d1, dims,
                                 preferred_element_type=jnp.float32)
        cw += jax.lax.dot_general(oh_bf, d2, dims,
                                  preferred_element_type=jnp.float32)
        cw_ref[:, c * DIM_EMBED:(c + 1) * DIM_EMBED] = cw
    for cp in oh_pending:
        cp.wait()


def kernel(mu, dictionary, ema_counts):
    del ema_counts
    batch, cw_dim = mu.shape
    cw, one_hot = pl.pallas_call(
        _vq_kernel,
        in_specs=[
            pl.BlockSpec(memory_space=pltpu.MemorySpace.VMEM),
            pl.BlockSpec(memory_space=pltpu.MemorySpace.HBM),
        ],
        out_specs=(
            pl.BlockSpec(memory_space=pltpu.MemorySpace.VMEM),
            pl.BlockSpec(memory_space=pltpu.MemorySpace.HBM),
        ),
        out_shape=(
            jax.ShapeDtypeStruct((batch, cw_dim), jnp.float32),
            jax.ShapeDtypeStruct((batch, DIM_CODES, DICT_SIZE), jnp.float32),
        ),
        scratch_shapes=[
            pltpu.MemorySpace.VMEM((2, DIM_CODES // 2, DICT_SIZE, DIM_EMBED),
                                   jnp.float32),
            pltpu.MemorySpace.VMEM((NBUF, BATCH, DICT_SIZE), jnp.float32),
            pltpu.SemaphoreType.DMA((2,)),
            pltpu.SemaphoreType.DMA((NBUF,)),
        ],
    )(mu, dictionary)
    return cw, one_hot


# speedup vs baseline: 1.0439x; 1.0439x over previous
import jax
import jax.numpy as jnp
from jax.experimental import pallas as pl
from jax.experimental.pallas import tpu as pltpu

BATCH = 256
DIM_CODES = 8
DICT_SIZE = 1024
DIM_EMBED = 64


NBUF = 4


def _vq_kernel(mu_ref, dict_ref, cw_ref, oh_hbm_ref, oh_buf, oh_sems):
    oh_pending = [None] * NBUF
    for c in range(DIM_CODES):
        x = mu_ref[:, c * DIM_EMBED:(c + 1) * DIM_EMBED]
        d = dict_ref[c]
        a2 = jnp.sum(x * x, axis=1, keepdims=True)
        b2 = jnp.sum(d * d, axis=1)[None, :]
        ab = jax.lax.dot_general(
            x, d, (((1,), (1,)), ((), ())),
            preferred_element_type=jnp.float32)
        dist = a2 - 2.0 * ab + b2
        idx = jnp.argmin(dist, axis=1)
        iota = jax.lax.broadcasted_iota(jnp.int32, (BATCH, DICT_SIZE), 1)
        hit = iota == idx[:, None].astype(jnp.int32)
        one_hot = hit.astype(jnp.float32)
        buf = c % NBUF
        if oh_pending[buf] is not None:
            oh_pending[buf].wait()
        oh_buf[buf] = one_hot
        cp = pltpu.make_async_copy(
            oh_buf.at[buf], oh_hbm_ref.at[:, c, :], oh_sems.at[buf])
        cp.start()
        oh_pending[buf] = cp
        oh_bf = hit.astype(jnp.bfloat16)
        d1 = d.astype(jnp.bfloat16)
        d2 = (d - d1.astype(jnp.float32)).astype(jnp.bfloat16)
        dims = (((1,), (0,)), ((), ()))
        cw = jax.lax.dot_general(oh_bf, d1, dims,
                                 preferred_element_type=jnp.float32)
        cw += jax.lax.dot_general(oh_bf, d2, dims,
                                  preferred_element_type=jnp.float32)
        cw_ref[:, c * DIM_EMBED:(c + 1) * DIM_EMBED] = cw
    for cp in oh_pending:
        cp.wait()


def kernel(mu, dictionary, ema_counts):
    del ema_counts
    batch, cw_dim = mu.shape
    cw, one_hot = pl.pallas_call(
        _vq_kernel,
        in_specs=[
            pl.BlockSpec(memory_space=pltpu.MemorySpace.VMEM),
            pl.BlockSpec(memory_space=pltpu.MemorySpace.VMEM),
        ],
        out_specs=(
            pl.BlockSpec(memory_space=pltpu.MemorySpace.VMEM),
            pl.BlockSpec(memory_space=pltpu.MemorySpace.HBM),
        ),
        out_shape=(
            jax.ShapeDtypeStruct((batch, cw_dim), jnp.float32),
            jax.ShapeDtypeStruct((batch, DIM_CODES, DICT_SIZE), jnp.float32),
        ),
        scratch_shapes=[
            pltpu.MemorySpace.VMEM((NBUF, BATCH, DICT_SIZE), jnp.float32),
            pltpu.SemaphoreType.DMA((NBUF,)),
        ],
    )(mu, dictionary)
    return cw, one_hot


# speedup vs baseline: 1.0534x; 1.0091x over previous
import jax
import jax.numpy as jnp
from jax.experimental import pallas as pl
from jax.experimental.pallas import tpu as pltpu

BATCH = 256
DIM_CODES = 8
DICT_SIZE = 1024
DIM_EMBED = 64


NBUF = 8


def _vq_kernel(mu_ref, dict_ref, cw_ref, oh_hbm_ref, oh_buf, oh_sems):
    oh_pending = [None] * NBUF
    for c in range(DIM_CODES):
        x = mu_ref[:, c * DIM_EMBED:(c + 1) * DIM_EMBED]
        d = dict_ref[c]
        a2 = jnp.sum(x * x, axis=1, keepdims=True)
        b2 = jnp.sum(d * d, axis=1)[None, :]
        ab = jax.lax.dot_general(
            x, d, (((1,), (1,)), ((), ())),
            preferred_element_type=jnp.float32)
        dist = a2 - 2.0 * ab + b2
        idx = jnp.argmin(dist, axis=1)
        iota = jax.lax.broadcasted_iota(jnp.int32, (BATCH, DICT_SIZE), 1)
        hit = iota == idx[:, None].astype(jnp.int32)
        one_hot = hit.astype(jnp.float32)
        buf = c % NBUF
        if oh_pending[buf] is not None:
            oh_pending[buf].wait()
        oh_buf[buf] = one_hot
        cp = pltpu.make_async_copy(
            oh_buf.at[buf], oh_hbm_ref.at[:, c, :], oh_sems.at[buf])
        cp.start()
        oh_pending[buf] = cp
        oh_bf = hit.astype(jnp.bfloat16)
        d1 = d.astype(jnp.bfloat16)
        d2 = (d - d1.astype(jnp.float32)).astype(jnp.bfloat16)
        dims = (((1,), (0,)), ((), ()))
        cw = jax.lax.dot_general(oh_bf, d1, dims,
                                 preferred_element_type=jnp.float32)
        cw += jax.lax.dot_general(oh_bf, d2, dims,
                                  preferred_element_type=jnp.float32)
        cw_ref[:, c * DIM_EMBED:(c + 1) * DIM_EMBED] = cw
    for cp in oh_pending:
        cp.wait()


def kernel(mu, dictionary, ema_counts):
    del ema_counts
    batch, cw_dim = mu.shape
    cw, one_hot = pl.pallas_call(
        _vq_kernel,
        in_specs=[
            pl.BlockSpec(memory_space=pltpu.MemorySpace.VMEM),
            pl.BlockSpec(memory_space=pltpu.MemorySpace.VMEM),
        ],
        out_specs=(
            pl.BlockSpec(memory_space=pltpu.MemorySpace.VMEM),
            pl.BlockSpec(memory_space=pltpu.MemorySpace.HBM),
        ),
        out_shape=(
            jax.ShapeDtypeStruct((batch, cw_dim), jnp.float32),
            jax.ShapeDtypeStruct((batch, DIM_CODES, DICT_SIZE), jnp.float32),
        ),
        scratch_shapes=[
            pltpu.MemorySpace.VMEM((NBUF, BATCH, DICT_SIZE), jnp.float32),
            pltpu.SemaphoreType.DMA((NBUF,)),
        ],
    )(mu, dictionary)
    return cw, one_hot
